# baseline (speedup 1.0000x reference)
import functools

import jax
import jax.numpy as jnp
import numpy as np
from jax import lax
from jax.experimental import pallas as pl
from jax.experimental.pallas import tpu as pltpu
from jax.experimental.pallas import tpu_sc as plsc

_NC = 2
_NS = 16
_L = 16
_NW = _NC * _NS
_B = 16384
_D = 32
_BPW = _B // _NW
_CHUNK = 128
_NCHUNK = _BPW // _CHUNK
_PAIR_UNROLL = 8


def _softplus(x):
    m = jnp.maximum(x, 0.0)
    e = jnp.exp(-jnp.abs(x))
    w = e / (2.0 + e)
    p = w * w
    series = 2.0 * w * (1.0 + p * (1.0 / 3.0 + p * (1.0 / 5.0
                        + p * (1.0 / 7.0 + p * (1.0 / 9.0)))))
    return m + series


def _vsqrt(d2):
    d2 = jnp.maximum(d2, 1e-30)
    i = lax.bitcast_convert_type(d2, jnp.int32)
    i = jnp.int32(0x5F3759DF) - lax.shift_right_logical(i, 1)
    r = lax.bitcast_convert_type(i, jnp.float32)
    r = r * (1.5 - 0.5 * d2 * r * r)
    r = r * (1.5 - 0.5 * d2 * r * r)
    return d2 * r


def _sc_body(uidx_hbm, vidx_hbm, lab_hbm, table_hbm, sinv_hbm, par_hbm,
             out_hbm,
             uidx_v, vidx_v, lab_v, urows_v, vrows_v, sinv_v, par_v,
             out_v, sem):
    wid = lax.axis_index("s") * _NC + lax.axis_index("c")
    base = wid * _BPW

    pltpu.sync_copy(uidx_hbm.at[wid], uidx_v)
    pltpu.sync_copy(vidx_hbm.at[wid], vidx_v)
    pltpu.sync_copy(lab_hbm.at[pl.ds(base, _BPW)], lab_v)
    pltpu.sync_copy(sinv_hbm, sinv_v)
    pltpu.sync_copy(par_hbm, par_v)

    copies = []
    for j in range(_NCHUNK):
        copies.append(pltpu.async_copy(
            table_hbm.at[uidx_v.at[j]],
            urows_v.at[pl.ds(j * _CHUNK, _CHUNK)], sem))
        copies.append(pltpu.async_copy(
            table_hbm.at[vidx_v.at[j]],
            vrows_v.at[pl.ds(j * _CHUNK, _CHUNK)], sem))
    for c in copies:
        c.wait()

    gam = par_v[pl.ds(0, _L)]
    c2 = par_v[pl.ds(_L, _L)]
    kq = 0.5 / (1000000.0 - 1.0)
    lane = lax.iota(jnp.int32, _L)
    s_lo = sinv_v[pl.ds(0, _L)]
    s_hi = sinv_v[pl.ds(_L, _L)]

    def group_body(g, _):
        d2 = jnp.zeros((_L,), jnp.float32)
        q = jnp.zeros((_L,), jnp.float32)
        for k in range(_L):
            p = g * _L + k
            u_lo = urows_v[p, pl.ds(0, _L)]
            u_hi = urows_v[p, pl.ds(_L, _L)]
            v_lo = vrows_v[p, pl.ds(0, _L)]
            v_hi = vrows_v[p, pl.ds(_L, _L)]
            t_lo = u_lo - v_lo
            t_hi = u_hi - v_hi
            hs = jnp.sum(t_lo * t_lo + t_hi * t_hi)
            ws = jnp.sum((u_lo * u_lo + v_lo * v_lo) * s_lo
                         + (u_hi * u_hi + v_hi * v_hi) * s_hi)
            sel = lane == k
            d2 = jnp.where(sel, hs, d2)
            q = jnp.where(sel, ws, q)
        lab = lab_v[pl.ds(g * _L, _L)]
        dist = _vsqrt(d2)
        x = jnp.where(lab == 1, dist - gam, gam - dist)
        out_v[pl.ds(g * _L, _L)] = _softplus(x) + c2 + kq * q
        return _

    lax.fori_loop(0, _BPW // _L, group_body, 0)

    pltpu.sync_copy(out_v, out_hbm.at[pl.ds(base, _BPW)])


@jax.jit
def _launch(uidx, vidx, labels, table, sinv, params):
    mesh = plsc.VectorSubcoreMesh(core_axis_name="c", subcore_axis_name="s")
    kern = functools.partial(
        pl.kernel,
        out_type=jax.ShapeDtypeStruct((_B,), jnp.float32),
        mesh=mesh,
        compiler_params=pltpu.CompilerParams(
            needs_layout_passes=False, use_tc_tiling_on_sc=False),
        scratch_types=[
            pltpu.VMEM((_NCHUNK, _CHUNK), jnp.int32),
            pltpu.VMEM((_NCHUNK, _CHUNK), jnp.int32),
            pltpu.VMEM((_BPW,), jnp.int32),
            pltpu.VMEM((_BPW, _D), jnp.float32),
            pltpu.VMEM((_BPW, _D), jnp.float32),
            pltpu.VMEM((_D,), jnp.float32),
            pltpu.VMEM((2 * _L,), jnp.float32),
            pltpu.VMEM((_BPW,), jnp.float32),
            pltpu.SemaphoreType.DMA,
        ],
    )(_sc_body)
    return kern(uidx, vidx, labels, table, sinv, params)


def kernel(pairs, labels, table, gamma, sigma):
    n_nodes, n_dim = table.shape
    uidx = pairs[:, 0].reshape(_NW, _NCHUNK, _CHUNK)
    vidx = pairs[:, 1].reshape(_NW, _NCHUNK, _CHUNK)
    sinv = (1.0 / sigma).astype(jnp.float32)
    const = (n_dim / 2.0) * np.float32(np.log(2.0 * np.pi)) \
        + 0.5 * jnp.sum(jnp.log(sigma))
    c2 = 2.0 * const / (n_nodes - 1)
    params = jnp.concatenate([
        jnp.full((_L,), gamma, dtype=jnp.float32),
        jnp.full((_L,), c2, dtype=jnp.float32),
    ])
    return _launch(uidx, vidx, labels.astype(jnp.int32), table, sinv, params)
